# baseline (speedup 1.0000x reference)
import jax
import jax.numpy as jnp
from jax.experimental import pallas as pl
from jax.experimental.pallas import tpu as pltpu

N_NODES = 10000
FEAT = 128
HID = 128
BM = 400
NB = N_NODES // BM


def _gcn_kernel(x_ref, adj_ref, W1_ref, b1_ref, W2_ref, b2_ref,
                rd_ref, fc1w_ref, fc1b_ref, out_ref, s1_ref, s2_ref):
    p = pl.program_id(0)
    j = pl.program_id(1)

    @pl.when(jnp.logical_and(p == 0, j == 0))
    def _init():
        s1_ref[...] = jnp.dot(x_ref[...], W1_ref[...],
                              preferred_element_type=jnp.float32)
        out_ref[...] = fc1b_ref[...]

    @pl.when(p == 0)
    def _phase0():
        h1 = jnp.dot(adj_ref[...], s1_ref[...],
                     preferred_element_type=jnp.float32)
        h1 = jnp.maximum(h1 + b1_ref[...], 0.0)
        s2_ref[pl.ds(j * BM, BM), :] = jnp.dot(
            h1, W2_ref[...], preferred_element_type=jnp.float32)

    @pl.when(p == 1)
    def _phase1():
        h2 = jnp.dot(adj_ref[...], s2_ref[...],
                     preferred_element_type=jnp.float32)
        h2 = jnp.maximum(h2 + b2_ref[...], 0.0)
        m = jnp.sum(h2, axis=1, keepdims=True) * (1.0 / HID)
        r = jnp.maximum(m * rd_ref[...], 0.0)
        out_ref[...] = out_ref[...] + jnp.sum(r * fc1w_ref[...])


def kernel(x, adj, W1, b1, W2, b2, rd_w, fc1_W, fc1_b):
    rd_col = rd_w.reshape(N_NODES, 1)
    fc1_col = fc1_W.reshape(N_NODES, 1)
    out = pl.pallas_call(
        _gcn_kernel,
        grid=(2, NB),
        in_specs=[
            pl.BlockSpec((N_NODES, FEAT), lambda p, j: (0, 0)),
            pl.BlockSpec((BM, N_NODES), lambda p, j: (j, 0)),
            pl.BlockSpec((FEAT, HID), lambda p, j: (0, 0)),
            pl.BlockSpec((1, HID), lambda p, j: (0, 0)),
            pl.BlockSpec((HID, HID), lambda p, j: (0, 0)),
            pl.BlockSpec((1, HID), lambda p, j: (0, 0)),
            pl.BlockSpec((BM, 1), lambda p, j: (j, 0)),
            pl.BlockSpec((BM, 1), lambda p, j: (j, 0)),
            pl.BlockSpec((1, 1), lambda p, j: (0, 0)),
        ],
        out_specs=pl.BlockSpec((1, 1), lambda p, j: (0, 0)),
        out_shape=jax.ShapeDtypeStruct((1, 1), jnp.float32),
        scratch_shapes=[
            pltpu.VMEM((N_NODES, HID), jnp.float32),
            pltpu.VMEM((N_NODES, HID), jnp.float32),
        ],
    )(x, adj, W1, b1.reshape(1, HID), W2, b2.reshape(1, HID),
      rd_col, fc1_col, fc1_b.reshape(1, 1))
    return out.reshape(1)
